# Initial kernel scaffold; baseline (speedup 1.0000x reference)
import jax
import jax.numpy as jnp
from jax.experimental import pallas as pl


def kernel(x, edge_index, W1l, b1, W1r, W2l, b2, W2r, W3l, b3, W3r):
    raise NotImplementedError("write your pallas kernel here")


# speedup vs baseline: 2.8647x; 2.8647x over previous
import jax
import jax.numpy as jnp
from jax import lax
from jax.experimental import pallas as pl
from jax.experimental.pallas import tpu as pltpu
from jax.experimental.pallas import tpu_sc as plsc

N = 10000
E = 320000
D = 128
DH = 64
NPAD = 10240
NS = 16
CHUNK = 80
EPT = E // NS
NCHUNKS = EPT // CHUNK
ZROWS = NPAD // NS
CW = 16


def _seg_sum_body(feat, src2, dst, zfeat, zcnt, ones,
                  out, cnt,
                  acc, cacc, idx_s, idx_d, rows, zv, zv16, ones_v, sem):
    c = lax.axis_index("c")
    s = lax.axis_index("s")

    pltpu.sync_copy(zfeat, zv)
    pltpu.sync_copy(zv, acc.at[pl.ds(s * ZROWS, ZROWS)])
    pltpu.sync_copy(zcnt, zv16)
    pltpu.sync_copy(zv16, cacc.at[pl.ds(s * ZROWS, ZROWS)])
    pltpu.sync_copy(ones, ones_v)
    plsc.subcore_barrier()

    base = s * EPT

    def step(k, _):
        eb = base + k * CHUNK
        pltpu.sync_copy(src2.at[pl.ds(c * E + eb, CHUNK)], idx_s)
        pltpu.sync_copy(dst.at[pl.ds(eb, CHUNK)], idx_d)
        pltpu.async_copy(feat.at[idx_s], rows, sem).wait()
        pltpu.sync_copy(rows, acc.at[idx_d], add=True)

        @pl.when(c == 0)
        def _():
            pltpu.sync_copy(ones_v, cacc.at[idx_d], add=True)

        return _

    lax.fori_loop(0, NCHUNKS, step, None)
    plsc.subcore_barrier()

    def _emit(nw):
        rb = s * ZROWS
        pltpu.sync_copy(acc.at[pl.ds(rb, nw)], zv.at[pl.ds(0, nw)])
        pltpu.sync_copy(zv.at[pl.ds(0, nw)], out.at[c, pl.ds(rb, nw)])

        @pl.when(c == 0)
        def _():
            pltpu.sync_copy(cacc.at[pl.ds(rb, nw)], zv16.at[pl.ds(0, nw)])
            pltpu.sync_copy(zv16.at[pl.ds(0, nw)], cnt.at[pl.ds(rb, nw)])

    @pl.when(s < NS - 1)
    def _():
        _emit(ZROWS)

    @pl.when(s == NS - 1)
    def _():
        _emit(N - (NS - 1) * ZROWS)


def _make_seg_sum():
    f32 = jnp.float32
    out_type = [jax.ShapeDtypeStruct((2, N, DH), f32),
                jax.ShapeDtypeStruct((N, CW), f32)]
    scratch = [
        pltpu.VMEM_SHARED((NPAD, DH), f32),
        pltpu.VMEM_SHARED((NPAD, CW), f32),
        pltpu.VMEM((CHUNK,), jnp.int32),
        pltpu.VMEM((CHUNK,), jnp.int32),
        pltpu.VMEM((CHUNK, DH), f32),
        pltpu.VMEM((ZROWS, DH), f32),
        pltpu.VMEM((ZROWS, CW), f32),
        pltpu.VMEM((CHUNK, CW), f32),
        pltpu.SemaphoreType.DMA,
    ]
    mesh = plsc.VectorSubcoreMesh(core_axis_name="c", subcore_axis_name="s")
    return pl.kernel(_seg_sum_body, out_type=out_type, mesh=mesh,
                     scratch_types=scratch,
                     compiler_params=pltpu.CompilerParams(
                         use_tc_tiling_on_sc=False))


_seg_sum = _make_seg_sum()

BN = 1000


def _l1_body(a, c, x, wl, b, wr, hlo, hhi):
    cnt = jnp.maximum(c[:, :1], 1.0)
    m0 = a[0] / cnt
    m1 = a[1] / cnt
    f32 = jnp.float32
    h = (jnp.dot(m0, wl[:DH, :], preferred_element_type=f32)
         + jnp.dot(m1, wl[DH:, :], preferred_element_type=f32)
         + jnp.dot(x[...], wr[...], preferred_element_type=f32)
         + b[...])
    h = jnp.where(h > 0, h, jnp.exp(h) - 1.0)
    hlo[0] = h[:, :DH]
    hlo[1] = h[:, DH:D]
    hhi[0] = h[:, D:D + DH]
    hhi[1] = h[:, D + DH:]


def _l23_body(alo, ahi, c, hlo, hhi,
              w2l, b2, w2r, w3l, b3, w3r, t_out, s_out):
    cnt = jnp.maximum(c[:, :1], 1.0)
    f32 = jnp.float32
    h = b2[...]
    for q, part in enumerate((alo[0] / cnt, alo[1] / cnt,
                              ahi[0] / cnt, ahi[1] / cnt)):
        h = h + jnp.dot(part, w2l[q * DH:(q + 1) * DH, :],
                        preferred_element_type=f32)
    for q, part in enumerate((hlo[0], hlo[1], hhi[0], hhi[1])):
        h = h + jnp.dot(part, w2r[q * DH:(q + 1) * DH, :],
                        preferred_element_type=f32)
    h = jnp.where(h > 0, h, jnp.exp(h) - 1.0)
    t = jnp.dot(h, w3l[...], preferred_element_type=f32)
    t_out[0] = t[:, :DH]
    t_out[1] = t[:, DH:]
    s_out[...] = jnp.dot(h, w3r[...], preferred_element_type=f32) + b3[...]


def _l3_body(t, c, s, out):
    cnt = jnp.maximum(c[:, :1], 1.0)
    z = jnp.concatenate([t[0], t[1]], axis=1) / cnt + s[...]
    z = jnp.where(z > 0, z, jnp.exp(z) - 1.0)
    m = jnp.max(z, axis=1, keepdims=True)
    lse = m + jnp.log(jnp.sum(jnp.exp(z - m), axis=1, keepdims=True))
    out[...] = z - lse


def _row_spec(cols):
    return pl.BlockSpec((BN, cols), lambda i: (i, 0))


def _stk_spec():
    return pl.BlockSpec((2, BN, DH), lambda i: (0, i, 0))


def _full_spec(shape):
    return pl.BlockSpec(shape, lambda i: tuple(0 for _ in shape))


def kernel(x, edge_index, W1l, b1, W1r, W2l, b2, W2r, W3l, b3, W3r):
    f32 = jnp.float32
    src = edge_index[0]
    dst = edge_index[1]
    src2 = jnp.concatenate([src, src + N])
    zfeat = jnp.zeros((ZROWS, DH), f32)
    zcnt = jnp.zeros((ZROWS, CW), f32)
    ones = jnp.ones((CHUNK, CW), f32)
    xs = jnp.stack([x[:, :DH], x[:, DH:]]).reshape(2 * N, DH)
    grid = (N // BN,)

    def run_l1(agg, cnt):
        return pl.pallas_call(
            _l1_body,
            grid=grid,
            in_specs=[_stk_spec(), _row_spec(CW),
                      _row_spec(D), _full_spec((D, 2 * D)),
                      _full_spec((1, 2 * D)), _full_spec((D, 2 * D))],
            out_specs=[_stk_spec(), _stk_spec()],
            out_shape=[jax.ShapeDtypeStruct((2, N, DH), f32),
                       jax.ShapeDtypeStruct((2, N, DH), f32)],
        )(agg, cnt, x, W1l, b1.reshape(1, -1), W1r)

    def run_l23(alo, ahi, cnt, hlo, hhi):
        return pl.pallas_call(
            _l23_body,
            grid=grid,
            in_specs=[_stk_spec(), _stk_spec(), _row_spec(CW),
                      _stk_spec(), _stk_spec(),
                      _full_spec((2 * D, 2 * D)), _full_spec((1, 2 * D)),
                      _full_spec((2 * D, 2 * D)), _full_spec((2 * D, D)),
                      _full_spec((1, D)), _full_spec((2 * D, D))],
            out_specs=[_stk_spec(), _row_spec(D)],
            out_shape=[jax.ShapeDtypeStruct((2, N, DH), f32),
                       jax.ShapeDtypeStruct((N, D), f32)],
        )(alo, ahi, cnt, hlo, hhi,
          W2l, b2.reshape(1, -1), W2r, W3l, b3.reshape(1, -1), W3r)

    def run_l3(tagg, cnt, sterm):
        return pl.pallas_call(
            _l3_body,
            grid=grid,
            in_specs=[_stk_spec(), _row_spec(CW), _row_spec(D)],
            out_specs=_row_spec(D),
            out_shape=jax.ShapeDtypeStruct((N, D), f32),
        )(tagg, cnt, sterm)

    def body(k, carry):
        feat, cnt, alo, hlo, hhi, sterm, result = carry
        agg, cnt_new = _seg_sum(feat, src2, dst, zfeat, zcnt, ones)

        def flat(a):
            return a.reshape(2 * N, DH)

        def b0(_):
            hlo_, hhi_ = run_l1(agg, cnt_new)
            return (flat(hlo_), cnt_new, alo, hlo_, hhi_, sterm, result)

        def b1(_):
            return (flat(hhi), cnt, agg, hlo, hhi, sterm, result)

        def b2(_):
            t, st = run_l23(alo, agg, cnt, hlo, hhi)
            return (flat(t), cnt, alo, hlo, hhi, st, result)

        def b3(_):
            res = run_l3(agg, cnt, sterm)
            return (feat, cnt, alo, hlo, hhi, sterm, res)

        return lax.switch(k, [b0, b1, b2, b3], None)

    zs = jnp.zeros((2, N, DH), f32)
    init = (xs, jnp.zeros((N, CW), f32), zs, zs, zs,
            jnp.zeros((N, D), f32), jnp.zeros((N, D), f32))
    final = lax.fori_loop(0, 4, body, init)
    return final[-1]
